# baseline (speedup 1.0000x reference)
import functools

import jax
import jax.numpy as jnp
from jax import lax
from jax.experimental import pallas as pl
from jax.experimental.pallas import tpu as pltpu
from jax.experimental.pallas import tpu_sc as plsc

N = 10000
NP = 10240
DUMMY = NP - 1
E = 320000
FIN = 128
FHID = 128
FOUT = 64
NC = 2
NS = 16
NW = NC * NS
K = 64
NCHUNK = 159
EPT = NCHUNK * K
ROWS_PT = NP // NS
DW = 8
RB = NP // 8
_MESH = plsc.VectorSubcoreMesh(core_axis_name="c", subcore_axis_name="s")
_SC_PARAMS = pltpu.CompilerParams(use_tc_tiling_on_sc=False)


@functools.partial(
    pl.kernel,
    out_type=[
        jax.ShapeDtypeStruct((NC, NP, DW), jnp.float32),
        jax.ShapeDtypeStruct((NC, NP, DW), jnp.float32),
    ],
    mesh=_MESH,
    compiler_params=_SC_PARAMS,
    scratch_types=[
        pltpu.VMEM((NCHUNK, K), jnp.int32),
        pltpu.VMEM((NCHUNK, K), jnp.int32),
        pltpu.VMEM((K, DW), jnp.float32),
        pltpu.VMEM_SHARED((NP, DW), jnp.float32),
        pltpu.VMEM_SHARED((NP, DW), jnp.float32),
        pltpu.SemaphoreType.DMA,
        pltpu.SemaphoreType.DMA,
        pltpu.SemaphoreType.DMA,
    ],
)
def _degrees(src_hbm, dst_hbm, ones_hbm, zeros_hbm, dego_hbm, degi_hbm,
             src_v, dst_v, ones_v, dego_s, degi_s, sem_i, sem_o, sem_d):
    c = lax.axis_index("c")
    s = lax.axis_index("s")
    wid = c * NS + s
    lo = s * ROWS_PT
    pltpu.async_copy(zeros_hbm, dego_s.at[pl.ds(lo, ROWS_PT)], sem_i)
    pltpu.async_copy(zeros_hbm, degi_s.at[pl.ds(lo, ROWS_PT)], sem_i)
    pltpu.async_copy(ones_hbm, ones_v, sem_i)
    pltpu.async_copy(src_hbm.at[wid], src_v, sem_i)
    pltpu.async_copy(dst_hbm.at[wid], dst_v, sem_i)
    pltpu.make_async_copy(zeros_hbm, dego_s.at[pl.ds(lo, ROWS_PT)], sem_i).wait()
    pltpu.make_async_copy(zeros_hbm, degi_s.at[pl.ds(lo, ROWS_PT)], sem_i).wait()
    pltpu.make_async_copy(ones_hbm, ones_v, sem_i).wait()
    pltpu.make_async_copy(src_hbm.at[wid], src_v, sem_i).wait()
    pltpu.make_async_copy(dst_hbm.at[wid], dst_v, sem_i).wait()
    plsc.subcore_barrier()

    @pl.loop(0, NCHUNK)
    def _(j):
        pltpu.sync_copy(ones_v, dego_s.at[src_v.at[j]], add=True)
        pltpu.sync_copy(ones_v, degi_s.at[dst_v.at[j]], add=True)

    plsc.subcore_barrier()
    pltpu.sync_copy(dego_s.at[pl.ds(lo, ROWS_PT)], dego_hbm.at[c, pl.ds(lo, ROWS_PT)])
    pltpu.sync_copy(degi_s.at[pl.ds(lo, ROWS_PT)], degi_hbm.at[c, pl.ds(lo, ROWS_PT)])


def _make_propagate(F):
    @functools.partial(
        pl.kernel,
        out_type=jax.ShapeDtypeStruct((NC, NP, F), jnp.float32),
        mesh=_MESH,
        compiler_params=_SC_PARAMS,
        scratch_types=[
            pltpu.VMEM((NCHUNK, K), jnp.int32),
            pltpu.VMEM((NCHUNK, K), jnp.int32),
            pltpu.VMEM((K, F), jnp.float32),
            pltpu.VMEM((K, F), jnp.float32),
            pltpu.VMEM((K, F), jnp.float32),
            pltpu.VMEM_SHARED((NP, F), jnp.float32),
            pltpu.SemaphoreType.DMA,
            pltpu.SemaphoreType.DMA,
            pltpu.SemaphoreType.DMA,
            pltpu.SemaphoreType.DMA,
            pltpu.SemaphoreType.DMA,
            pltpu.SemaphoreType.DMA,
            pltpu.SemaphoreType.DMA,
        ],
    )
    def _propagate(h_hbm, src_hbm, dst_hbm, zeros_hbm, out_hbm,
                   src_v, dst_v, r0, r1, r2, agg,
                   g0, g1, g2, s0, s1, s2, sem_i):
        c = lax.axis_index("c")
        s = lax.axis_index("s")
        wid = c * NS + s
        lo = s * ROWS_PT
        rows = (r0, r1, r2)
        gsem = (g0, g1, g2)
        ssem = (s0, s1, s2)

        pltpu.async_copy(zeros_hbm, agg.at[pl.ds(lo, ROWS_PT)], sem_i)
        pltpu.async_copy(src_hbm.at[wid], src_v, sem_i)
        pltpu.async_copy(dst_hbm.at[wid], dst_v, sem_i)
        pltpu.make_async_copy(zeros_hbm, agg.at[pl.ds(lo, ROWS_PT)], sem_i).wait()
        pltpu.make_async_copy(src_hbm.at[wid], src_v, sem_i).wait()
        pltpu.make_async_copy(dst_hbm.at[wid], dst_v, sem_i).wait()

        def fire_g(j, u):
            pltpu.async_copy(h_hbm.at[src_v.at[j]], rows[u], gsem[u])

        def wait_g(j, u):
            pltpu.make_async_copy(h_hbm.at[src_v.at[j]], rows[u], gsem[u]).wait()

        def fire_s(j, u):
            pltpu.async_copy(rows[u], agg.at[dst_v.at[j]], ssem[u], add=True)

        def wait_s(j, u):
            pltpu.make_async_copy(rows[u], agg.at[dst_v.at[j]], ssem[u]).wait()

        fire_g(0, 0)
        fire_g(1, 1)
        plsc.subcore_barrier()

        @pl.loop(0, NCHUNK // 3)
        def _(grp):
            for u in range(3):
                j = 3 * grp + u

                wait_g(j, u)
                v = (u + 2) % 3

                @pl.when(j <= NCHUNK - 3)
                def _():
                    fire_g(j + 2, v)

                pltpu.sync_copy(rows[u], agg.at[dst_v.at[j]], add=True)

        plsc.subcore_barrier()
        pltpu.sync_copy(agg.at[pl.ds(lo, ROWS_PT)], out_hbm.at[c, pl.ds(lo, ROWS_PT)])

    return _propagate


_prop_hid = _make_propagate(FHID)
_prop_out = _make_propagate(FOUT)


def _norms_body(x_ref, dgo_ref, dgi_ref, h0_ref, no_ref, ni_ref):
    dgo = (dgo_ref[0] + dgo_ref[1])[:, 0:1]
    dgi = (dgi_ref[0] + dgi_ref[1])[:, 0:1]
    no = jnp.where(dgo > 0, lax.rsqrt(jnp.maximum(dgo, 1.0)), 0.0)
    ni = jnp.where(dgi > 0, lax.rsqrt(jnp.maximum(dgi, 1.0)), 0.0)
    no_ref[...] = no
    ni_ref[...] = ni
    h0_ref[...] = x_ref[...] * no


_norms_call = pl.pallas_call(
    _norms_body,
    grid=(NP // RB,),
    in_specs=[
        pl.BlockSpec((RB, FIN), lambda i: (i, 0)),
        pl.BlockSpec((NC, RB, DW), lambda i: (0, i, 0)),
        pl.BlockSpec((NC, RB, DW), lambda i: (0, i, 0)),
    ],
    out_specs=[
        pl.BlockSpec((RB, FIN), lambda i: (i, 0)),
        pl.BlockSpec((RB, 1), lambda i: (i, 0)),
        pl.BlockSpec((RB, 1), lambda i: (i, 0)),
    ],
    out_shape=[
        jax.ShapeDtypeStruct((NP, FIN), jnp.float32),
        jax.ShapeDtypeStruct((NP, 1), jnp.float32),
        jax.ShapeDtypeStruct((NP, 1), jnp.float32),
    ],
)


def _dense_body(p_ref, ni_ref, no_ref, w1_ref, b1_ref, w2_ref, t_ref):
    agg = (p_ref[0] + p_ref[1]) * ni_ref[...]
    h1 = jnp.dot(agg, w1_ref[...], preferred_element_type=jnp.float32)
    h1 = jnp.maximum(h1 + b1_ref[...], 0.0)
    t = jnp.dot(h1, w2_ref[...], preferred_element_type=jnp.float32)
    t_ref[...] = t * no_ref[...]


_dense_call = pl.pallas_call(
    _dense_body,
    grid=(NP // RB,),
    in_specs=[
        pl.BlockSpec((NC, RB, FHID), lambda i: (0, i, 0)),
        pl.BlockSpec((RB, 1), lambda i: (i, 0)),
        pl.BlockSpec((RB, 1), lambda i: (i, 0)),
        pl.BlockSpec((FIN, FHID), lambda i: (0, 0)),
        pl.BlockSpec((1, FHID), lambda i: (0, 0)),
        pl.BlockSpec((FHID, FOUT), lambda i: (0, 0)),
    ],
    out_specs=pl.BlockSpec((RB, FOUT), lambda i: (i, 0)),
    out_shape=jax.ShapeDtypeStruct((NP, FOUT), jnp.float32),
)


def _final_body(q_ref, ni_ref, b2_ref, out_ref):
    out_ref[...] = (q_ref[0] + q_ref[1]) * ni_ref[...] + b2_ref[...]


_final_call = pl.pallas_call(
    _final_body,
    grid=(NP // RB,),
    in_specs=[
        pl.BlockSpec((NC, RB, FOUT), lambda i: (0, i, 0)),
        pl.BlockSpec((RB, 1), lambda i: (i, 0)),
        pl.BlockSpec((1, FOUT), lambda i: (0, 0)),
    ],
    out_specs=pl.BlockSpec((RB, FOUT), lambda i: (i, 0)),
    out_shape=jax.ShapeDtypeStruct((NP, FOUT), jnp.float32),
)


def _pad_edges(e):
    e = e.astype(jnp.int32).reshape(NW, E // NW)
    pad = jnp.full((NW, EPT - E // NW), DUMMY, jnp.int32)
    return jnp.concatenate([e, pad], axis=1).reshape(NW, NCHUNK, K)


def kernel(inputs, edge_index, W1, b1, W2, b2):
    src = _pad_edges(edge_index[0])
    dst = _pad_edges(edge_index[1])
    x_pad = jnp.pad(inputs, ((0, NP - N), (0, 0)))
    ones_kw = jnp.ones((K, DW), jnp.float32)
    zeros_dw = jnp.zeros((ROWS_PT, DW), jnp.float32)
    zeros_hid = jnp.zeros((ROWS_PT, FHID), jnp.float32)
    zeros_out = jnp.zeros((ROWS_PT, FOUT), jnp.float32)
    dego, degi = _degrees(src, dst, ones_kw, zeros_dw)
    h0, no, ni = _norms_call(x_pad, dego, degi)
    p = _prop_hid(h0, src, dst, zeros_hid)
    t = _dense_call(p, ni, no, W1, b1.reshape(1, FHID), W2)
    q = _prop_out(t, src, dst, zeros_out)
    out = _final_call(q, ni, b2.reshape(1, FOUT))
    return out[:N]


# speedup vs baseline: 1.0291x; 1.0291x over previous
import functools

import jax
import jax.numpy as jnp
from jax import lax
from jax.experimental import pallas as pl
from jax.experimental.pallas import tpu as pltpu
from jax.experimental.pallas import tpu_sc as plsc

N = 10000
NP = 10240
DUMMY = NP - 1
E = 320000
FIN = 128
FHID = 128
FOUT = 64
NC = 2
NS = 16
NW = NC * NS
K = 64
NCHUNK = 159
EPT = NCHUNK * K
ROWS_PT = NP // NS
DW = 8
RB = NP // 8
_MESH = plsc.VectorSubcoreMesh(core_axis_name="c", subcore_axis_name="s")
_SC_PARAMS = pltpu.CompilerParams(use_tc_tiling_on_sc=False)


@functools.partial(
    pl.kernel,
    out_type=[
        jax.ShapeDtypeStruct((NC, NP, DW), jnp.float32),
        jax.ShapeDtypeStruct((NC, NP, DW), jnp.float32),
    ],
    mesh=_MESH,
    compiler_params=_SC_PARAMS,
    scratch_types=[
        pltpu.VMEM((NCHUNK, K), jnp.int32),
        pltpu.VMEM((NCHUNK, K), jnp.int32),
        pltpu.VMEM((K, DW), jnp.float32),
        pltpu.VMEM_SHARED((NP, DW), jnp.float32),
        pltpu.VMEM_SHARED((NP, DW), jnp.float32),
        pltpu.SemaphoreType.DMA,
        pltpu.SemaphoreType.DMA,
        pltpu.SemaphoreType.DMA,
    ],
)
def _degrees(src_hbm, dst_hbm, ones_hbm, zeros_hbm, dego_hbm, degi_hbm,
             src_v, dst_v, ones_v, dego_s, degi_s, sem_i, sem_o, sem_d):
    c = lax.axis_index("c")
    s = lax.axis_index("s")
    wid = c * NS + s
    lo = s * ROWS_PT
    pltpu.async_copy(zeros_hbm, dego_s.at[pl.ds(lo, ROWS_PT)], sem_i)
    pltpu.async_copy(zeros_hbm, degi_s.at[pl.ds(lo, ROWS_PT)], sem_i)
    pltpu.async_copy(ones_hbm, ones_v, sem_i)
    pltpu.async_copy(src_hbm.at[wid], src_v, sem_i)
    pltpu.async_copy(dst_hbm.at[wid], dst_v, sem_i)
    pltpu.make_async_copy(zeros_hbm, dego_s.at[pl.ds(lo, ROWS_PT)], sem_i).wait()
    pltpu.make_async_copy(zeros_hbm, degi_s.at[pl.ds(lo, ROWS_PT)], sem_i).wait()
    pltpu.make_async_copy(ones_hbm, ones_v, sem_i).wait()
    pltpu.make_async_copy(src_hbm.at[wid], src_v, sem_i).wait()
    pltpu.make_async_copy(dst_hbm.at[wid], dst_v, sem_i).wait()
    plsc.subcore_barrier()

    @pl.loop(0, NCHUNK)
    def _(j):
        pltpu.async_copy(ones_v, dego_s.at[src_v.at[j]], sem_o, add=True)
        pltpu.async_copy(ones_v, degi_s.at[dst_v.at[j]], sem_d, add=True)

        @pl.when(j >= 1)
        def _():
            pltpu.make_async_copy(ones_v, dego_s.at[src_v.at[j - 1]], sem_o).wait()
            pltpu.make_async_copy(ones_v, degi_s.at[dst_v.at[j - 1]], sem_d).wait()

    pltpu.make_async_copy(ones_v, dego_s.at[src_v.at[NCHUNK - 1]], sem_o).wait()
    pltpu.make_async_copy(ones_v, degi_s.at[dst_v.at[NCHUNK - 1]], sem_d).wait()
    plsc.subcore_barrier()
    pltpu.sync_copy(dego_s.at[pl.ds(lo, ROWS_PT)], dego_hbm.at[c, pl.ds(lo, ROWS_PT)])
    pltpu.sync_copy(degi_s.at[pl.ds(lo, ROWS_PT)], degi_hbm.at[c, pl.ds(lo, ROWS_PT)])


def _make_propagate(F):
    @functools.partial(
        pl.kernel,
        out_type=jax.ShapeDtypeStruct((NC, NP, F), jnp.float32),
        mesh=_MESH,
        compiler_params=_SC_PARAMS,
        scratch_types=[
            pltpu.VMEM((NCHUNK, K), jnp.int32),
            pltpu.VMEM((NCHUNK, K), jnp.int32),
            pltpu.VMEM((K, F), jnp.float32),
            pltpu.VMEM((K, F), jnp.float32),
            pltpu.VMEM((K, F), jnp.float32),
            pltpu.VMEM_SHARED((NP, F), jnp.float32),
            pltpu.SemaphoreType.DMA,
            pltpu.SemaphoreType.DMA,
            pltpu.SemaphoreType.DMA,
            pltpu.SemaphoreType.DMA,
            pltpu.SemaphoreType.DMA,
            pltpu.SemaphoreType.DMA,
            pltpu.SemaphoreType.DMA,
        ],
    )
    def _propagate(h_hbm, src_hbm, dst_hbm, zeros_hbm, out_hbm,
                   src_v, dst_v, r0, r1, r2, agg,
                   g0, g1, g2, s0, s1, s2, sem_i):
        c = lax.axis_index("c")
        s = lax.axis_index("s")
        wid = c * NS + s
        lo = s * ROWS_PT
        rows = (r0, r1, r2)
        gsem = (g0, g1, g2)
        ssem = (s0, s1, s2)

        pltpu.async_copy(zeros_hbm, agg.at[pl.ds(lo, ROWS_PT)], sem_i)
        pltpu.async_copy(src_hbm.at[wid], src_v, sem_i)
        pltpu.async_copy(dst_hbm.at[wid], dst_v, sem_i)
        pltpu.make_async_copy(zeros_hbm, agg.at[pl.ds(lo, ROWS_PT)], sem_i).wait()
        pltpu.make_async_copy(src_hbm.at[wid], src_v, sem_i).wait()
        pltpu.make_async_copy(dst_hbm.at[wid], dst_v, sem_i).wait()

        def fire_g(j, u):
            pltpu.async_copy(h_hbm.at[src_v.at[j]], rows[u], gsem[u])

        def wait_g(j, u):
            pltpu.make_async_copy(h_hbm.at[src_v.at[j]], rows[u], gsem[u]).wait()

        def fire_s(j, u):
            pltpu.async_copy(rows[u], agg.at[dst_v.at[j]], ssem[u], add=True)

        def wait_s(j, u):
            pltpu.make_async_copy(rows[u], agg.at[dst_v.at[j]], ssem[u]).wait()

        fire_g(0, 0)
        fire_g(1, 1)
        plsc.subcore_barrier()

        @pl.loop(0, NCHUNK // 3)
        def _(grp):
            for u in range(3):
                j = 3 * grp + u

                wait_g(j, u)
                fire_s(j, u)
                v = (u + 2) % 3

                @pl.when((j >= 1) & (j <= NCHUNK - 3))
                def _():
                    wait_s(j - 1, v)

                @pl.when(j <= NCHUNK - 3)
                def _():
                    fire_g(j + 2, v)

        for u in range(3):
            wait_s(NCHUNK - 3 + u, u)
        plsc.subcore_barrier()
        pltpu.sync_copy(agg.at[pl.ds(lo, ROWS_PT)], out_hbm.at[c, pl.ds(lo, ROWS_PT)])

    return _propagate


_prop_hid = _make_propagate(FHID)
_prop_out = _make_propagate(FOUT)


def _norms_body(x_ref, dgo_ref, dgi_ref, h0_ref, no_ref, ni_ref):
    dgo = (dgo_ref[0] + dgo_ref[1])[:, 0:1]
    dgi = (dgi_ref[0] + dgi_ref[1])[:, 0:1]
    no = jnp.where(dgo > 0, lax.rsqrt(jnp.maximum(dgo, 1.0)), 0.0)
    ni = jnp.where(dgi > 0, lax.rsqrt(jnp.maximum(dgi, 1.0)), 0.0)
    no_ref[...] = no
    ni_ref[...] = ni
    h0_ref[...] = x_ref[...] * no


_norms_call = pl.pallas_call(
    _norms_body,
    grid=(NP // RB,),
    in_specs=[
        pl.BlockSpec((RB, FIN), lambda i: (i, 0)),
        pl.BlockSpec((NC, RB, DW), lambda i: (0, i, 0)),
        pl.BlockSpec((NC, RB, DW), lambda i: (0, i, 0)),
    ],
    out_specs=[
        pl.BlockSpec((RB, FIN), lambda i: (i, 0)),
        pl.BlockSpec((RB, 1), lambda i: (i, 0)),
        pl.BlockSpec((RB, 1), lambda i: (i, 0)),
    ],
    out_shape=[
        jax.ShapeDtypeStruct((NP, FIN), jnp.float32),
        jax.ShapeDtypeStruct((NP, 1), jnp.float32),
        jax.ShapeDtypeStruct((NP, 1), jnp.float32),
    ],
)


def _dense_body(p_ref, ni_ref, no_ref, w1_ref, b1_ref, w2_ref, t_ref):
    agg = (p_ref[0] + p_ref[1]) * ni_ref[...]
    h1 = jnp.dot(agg, w1_ref[...], preferred_element_type=jnp.float32)
    h1 = jnp.maximum(h1 + b1_ref[...], 0.0)
    t = jnp.dot(h1, w2_ref[...], preferred_element_type=jnp.float32)
    t_ref[...] = t * no_ref[...]


_dense_call = pl.pallas_call(
    _dense_body,
    grid=(NP // RB,),
    in_specs=[
        pl.BlockSpec((NC, RB, FHID), lambda i: (0, i, 0)),
        pl.BlockSpec((RB, 1), lambda i: (i, 0)),
        pl.BlockSpec((RB, 1), lambda i: (i, 0)),
        pl.BlockSpec((FIN, FHID), lambda i: (0, 0)),
        pl.BlockSpec((1, FHID), lambda i: (0, 0)),
        pl.BlockSpec((FHID, FOUT), lambda i: (0, 0)),
    ],
    out_specs=pl.BlockSpec((RB, FOUT), lambda i: (i, 0)),
    out_shape=jax.ShapeDtypeStruct((NP, FOUT), jnp.float32),
)


def _final_body(q_ref, ni_ref, b2_ref, out_ref):
    out_ref[...] = (q_ref[0] + q_ref[1]) * ni_ref[...] + b2_ref[...]


_final_call = pl.pallas_call(
    _final_body,
    grid=(NP // RB,),
    in_specs=[
        pl.BlockSpec((NC, RB, FOUT), lambda i: (0, i, 0)),
        pl.BlockSpec((RB, 1), lambda i: (i, 0)),
        pl.BlockSpec((1, FOUT), lambda i: (0, 0)),
    ],
    out_specs=pl.BlockSpec((RB, FOUT), lambda i: (i, 0)),
    out_shape=jax.ShapeDtypeStruct((NP, FOUT), jnp.float32),
)


def _pad_edges(e):
    e = e.astype(jnp.int32).reshape(NW, E // NW)
    pad = jnp.full((NW, EPT - E // NW), DUMMY, jnp.int32)
    return jnp.concatenate([e, pad], axis=1).reshape(NW, NCHUNK, K)


def kernel(inputs, edge_index, W1, b1, W2, b2):
    src = _pad_edges(edge_index[0])
    dst = _pad_edges(edge_index[1])
    x_pad = jnp.pad(inputs, ((0, NP - N), (0, 0)))
    ones_kw = jnp.ones((K, DW), jnp.float32)
    zeros_dw = jnp.zeros((ROWS_PT, DW), jnp.float32)
    zeros_hid = jnp.zeros((ROWS_PT, FHID), jnp.float32)
    zeros_out = jnp.zeros((ROWS_PT, FOUT), jnp.float32)
    dego, degi = _degrees(src, dst, ones_kw, zeros_dw)
    h0, no, ni = _norms_call(x_pad, dego, degi)
    p = _prop_hid(h0, src, dst, zeros_hid)
    t = _dense_call(p, ni, no, W1, b1.reshape(1, FHID), W2)
    q = _prop_out(t, src, dst, zeros_out)
    out = _final_call(q, ni, b2.reshape(1, FOUT))
    return out[:N]
